# baseline (speedup 1.0000x reference)
import numpy as np
import jax
import jax.numpy as jnp
from jax import lax
from jax.experimental import pallas as pl
from jax.experimental.pallas import tpu as pltpu
from jax.experimental.pallas import tpu_sc as plsc

_B = 4
_N = 16384
_C = 128
_NPOINT = 1024
_NSAMPLE = 64
_RADIUS = np.float32(0.4)
_R2 = np.float32(0.4 * 0.4)

_SIDE = 128

_NW = 32
_SPB = _NW // _B
_SEEDS_PER_W = _NPOINT // _SPB
_NCHUNK = _N // 16
_UNROLL = 8


def _fps_body(xyzt_ref, inds_ref, dists_ref):
    X = xyzt_ref[:, 0, :, :]
    Y = xyzt_ref[:, 1, :, :]
    Z = xyzt_ref[:, 2, :, :]
    row = lax.broadcasted_iota(jnp.int32, (_SIDE, _SIDE), 0)
    col = lax.broadcasted_iota(jnp.int32, (_SIDE, _SIDE), 1)
    iota2 = (row * _SIDE + col)[None]
    cols = lax.broadcasted_iota(jnp.int32, (1, _NPOINT), 1)
    l128 = lax.broadcasted_iota(jnp.int32, (1, _SIDE), 1)
    big = jnp.int32(1 << 30)
    dists_ref[...] = jnp.full((_B, _SIDE, _SIDE), 1e10, dtype=jnp.float32)

    def body(i, state):
        fara, farb, inds = state
        farcat = jnp.concatenate([fara, farb], axis=0).reshape(_B, 1)
        inds = inds + jnp.where(cols == i, farcat, 0)
        new_fars = []
        for g, far in ((0, fara), (1, farb)):
            sl = slice(2 * g, 2 * g + 2)
            Xg = X[sl]
            Yg = Y[sl]
            Zg = Z[sl]
            oh = iota2 == far
            cxyz = jnp.sum(jnp.where(oh[:, None], xyzt_ref[sl], 0.0),
                           axis=(2, 3), keepdims=True)
            cx = cxyz[:, 0]
            cy = cxyz[:, 1]
            cz = cxyz[:, 2]
            dx = Xg - cx
            dy = Yg - cy
            dz = Zg - cz
            d = dx * dx + dy * dy
            d = d + dz * dz
            dists = jnp.minimum(dists_ref[sl], d)
            dists_ref[sl] = dists
            m = jnp.max(dists, axis=(1, 2), keepdims=True)
            fnew = jnp.min(jnp.where(dists == m, iota2, big),
                           axis=(1, 2), keepdims=True)
            new_fars.append(fnew.astype(jnp.int32))
        return (new_fars[0], new_fars[1], inds)

    far0 = jnp.zeros((2, 1, 1), dtype=jnp.int32)
    inds0 = jnp.zeros((_B, _NPOINT), dtype=jnp.int32)
    _, _, inds_final = lax.fori_loop(0, _NPOINT, body, (far0, far0, inds0))
    inds_ref[...] = inds_final


def _sc_group_body(xyzt_hbm, inds_hbm, feat_hbm,
                   newxyz_hbm, gxyz_hbm, gfeat_hbm,
                   xv, yv, zv, indsv, sxv, syv, szv, nxv,
                   grpbuf, gidxa, gidxb, xyza, xyzb, rowa, rowb, sem, osem):
    cid = lax.axis_index("c")
    sid = lax.axis_index("s")
    wid = sid * 2 + cid
    b = wid // _SPB
    s0 = (wid % _SPB) * _SEEDS_PER_W
    bN = b * _N

    pltpu.sync_copy(xyzt_hbm.at[pl.ds((b * 3 + 0) * _N, _N)], xv)
    pltpu.sync_copy(xyzt_hbm.at[pl.ds((b * 3 + 1) * _N, _N)], yv)
    pltpu.sync_copy(xyzt_hbm.at[pl.ds((b * 3 + 2) * _N, _N)], zv)
    pltpu.sync_copy(inds_hbm.at[pl.ds(b * _NPOINT + s0, _SEEDS_PER_W)], indsv)

    iota16 = lax.broadcasted_iota(jnp.int32, (16,), 0)
    c0 = jnp.zeros((16,), jnp.int32)
    c1 = jnp.full((16,), 1, jnp.int32)
    c2 = jnp.full((16,), 2, jnp.int32)

    sxv[pl.ds(_SEEDS_PER_W, 16)] = jnp.zeros((16,), jnp.float32)
    syv[pl.ds(_SEEDS_PER_W, 16)] = jnp.zeros((16,), jnp.float32)
    szv[pl.ds(_SEEDS_PER_W, 16)] = jnp.zeros((16,), jnp.float32)

    for q in range(_SEEDS_PER_W // 16):
        iv = indsv[pl.ds(q * 16, 16)]
        sx = plsc.load_gather(xv, [iv])
        sy = plsc.load_gather(yv, [iv])
        sz = plsc.load_gather(zv, [iv])
        sxv[pl.ds(q * 16, 16)] = sx
        syv[pl.ds(q * 16, 16)] = sy
        szv[pl.ds(q * 16, 16)] = sz
        kvec3 = (iota16 + q * 16) * 3
        plsc.store_scatter(nxv, [kvec3 + c0], sx)
        plsc.store_scatter(nxv, [kvec3 + c1], sy)
        plsc.store_scatter(nxv, [kvec3 + c2], sz)
    pltpu.sync_copy(nxv, newxyz_hbm.at[pl.ds((b * _NPOINT + s0) * 3, _SEEDS_PER_W * 3)])

    def build(t, gidx_ref, xyzb_ref):
        qb = (t // 16) * 16
        lane = t - qb
        lm = iota16 == lane
        sxt = jnp.sum(jnp.where(lm, sxv[pl.ds(qb, 16)], 0.0))
        syt = jnp.sum(jnp.where(lm, syv[pl.ds(qb, 16)], 0.0))
        szt = jnp.sum(jnp.where(lm, szv[pl.ds(qb, 16)], 0.0))

        def cond(st):
            j, offv = st
            return jnp.logical_and(j < _NCHUNK,
                                   jnp.all(offv < _NSAMPLE))

        def sbody(st):
            j, offv = st
            acc = offv - 1
            for u in range(_UNROLL):
                base = (j + u) * 16
                xs = xv[pl.ds(base, 16)]
                ys = yv[pl.ds(base, 16)]
                zs = zv[pl.ds(base, 16)]
                dx = xs - sxt
                dy = ys - syt
                dz = zs - szt
                d = dx * dx + dy * dy
                d = d + dz * dz
                m = d <= _R2
                pos = plsc.cumsum(m.astype(jnp.int32)) + acc
                plsc.store_scatter(grpbuf, [pos], iota16 + base, mask=m)
                acc = acc + plsc.all_reduce_population_count(m)
            return (j + _UNROLL, acc + 1)

        _, offv = lax.while_loop(
            cond, sbody, (jnp.int32(0), jnp.zeros((16,), jnp.int32)))
        total = jnp.max(offv)

        f16 = grpbuf[pl.ds(0, 16)]
        first = jnp.sum(jnp.where(iota16 == 0, f16, 0))

        for c in range(_NSAMPLE // 16):
            kvec = iota16 + c * 16
            g = grpbuf[pl.ds(c * 16, 16)]
            g = jnp.where(kvec < total, g, first)
            gx = plsc.load_gather(xv, [g])
            gy = plsc.load_gather(yv, [g])
            gz = plsc.load_gather(zv, [g])
            kvec3 = kvec * 3
            plsc.store_scatter(xyzb_ref, [kvec3 + c0], (gx - sxt) / _RADIUS)
            plsc.store_scatter(xyzb_ref, [kvec3 + c1], (gy - syt) / _RADIUS)
            plsc.store_scatter(xyzb_ref, [kvec3 + c2], (gz - szt) / _RADIUS)
            gidx_ref[pl.ds(c * 16, 16)] = g + bN

    def feat_dst(t):
        s = b * _NPOINT + s0 + t
        return gfeat_hbm.at[pl.ds(s * _NSAMPLE, _NSAMPLE), :]

    def xyz_out(t, xyzb_ref):
        s = b * _NPOINT + s0 + t
        pltpu.sync_copy(xyzb_ref, gxyz_hbm.at[pl.ds(s * _NSAMPLE * 3, _NSAMPLE * 3)])

    build(0, gidxa, xyza)
    pltpu.async_copy(feat_hbm.at[gidxa], rowa, sem)
    pltpu.async_copy(feat_hbm.at[pl.ds(0, _NSAMPLE), :], rowb, osem)

    def pair(i, carry):
        t0 = 2 * i
        build(t0 + 1, gidxb, xyzb)
        pltpu.make_async_copy(feat_hbm.at[gidxa], rowa, sem).wait()
        pltpu.make_async_copy(rowb, feat_dst(t0 + 1), osem).wait()
        pltpu.async_copy(feat_hbm.at[gidxb], rowb, sem)
        pltpu.async_copy(rowa, feat_dst(t0), osem)
        xyz_out(t0, xyza)
        build(t0 + 2, gidxa, xyza)
        pltpu.make_async_copy(feat_hbm.at[gidxb], rowb, sem).wait()
        pltpu.make_async_copy(rowa, feat_dst(t0), osem).wait()
        pltpu.async_copy(feat_hbm.at[gidxa], rowa, sem)
        pltpu.async_copy(rowb, feat_dst(t0 + 1), osem)
        xyz_out(t0 + 1, xyzb)
        return carry

    lax.fori_loop(0, _SEEDS_PER_W // 2, pair, jnp.int32(0))
    pltpu.make_async_copy(rowb, feat_dst(_SEEDS_PER_W - 1), osem).wait()
    pltpu.make_async_copy(feat_hbm.at[gidxa], rowa, sem).wait()


def kernel(xyz, isPainted, features):
    xyzt = xyz.transpose(0, 2, 1).reshape(_B, 3, _SIDE, _SIDE)

    inds = pl.pallas_call(
        _fps_body,
        out_shape=jax.ShapeDtypeStruct((_B, _NPOINT), jnp.int32),
        scratch_shapes=[pltpu.VMEM((_B, _SIDE, _SIDE), jnp.float32)],
    )(xyzt)

    xyztflat = xyzt.reshape(_B * 3 * _N)
    indsflat = inds.reshape(_B * _NPOINT)
    featflat = features.reshape(_B * _N, _C)

    mesh = plsc.VectorSubcoreMesh(core_axis_name="c", subcore_axis_name="s")
    newxyz, gxyz, gfeat = pl.kernel(
        _sc_group_body,
        out_type=[
            jax.ShapeDtypeStruct((_B * _NPOINT * 3,), jnp.float32),
            jax.ShapeDtypeStruct((_B * _NPOINT * _NSAMPLE * 3,), jnp.float32),
            jax.ShapeDtypeStruct((_B * _NPOINT * _NSAMPLE, _C), jnp.float32),
        ],
        mesh=mesh,
        compiler_params=pltpu.CompilerParams(needs_layout_passes=False),
        scratch_types=[
            pltpu.VMEM((_N,), jnp.float32),
            pltpu.VMEM((_N,), jnp.float32),
            pltpu.VMEM((_N,), jnp.float32),
            pltpu.VMEM((_SEEDS_PER_W,), jnp.int32),
            pltpu.VMEM((_SEEDS_PER_W + 16,), jnp.float32),
            pltpu.VMEM((_SEEDS_PER_W + 16,), jnp.float32),
            pltpu.VMEM((_SEEDS_PER_W + 16,), jnp.float32),
            pltpu.VMEM((_SEEDS_PER_W * 3,), jnp.float32),
            pltpu.VMEM((192,), jnp.int32),
            pltpu.VMEM((_NSAMPLE,), jnp.int32),
            pltpu.VMEM((_NSAMPLE,), jnp.int32),
            pltpu.VMEM((_NSAMPLE * 3,), jnp.float32),
            pltpu.VMEM((_NSAMPLE * 3,), jnp.float32),
            pltpu.VMEM((_NSAMPLE, _C), jnp.float32),
            pltpu.VMEM((_NSAMPLE, _C), jnp.float32),
            pltpu.SemaphoreType.DMA,
            pltpu.SemaphoreType.DMA,
        ],
    )(xyztflat, indsflat, featflat)

    newxyz = newxyz.reshape(_B, _NPOINT, 3)
    gxyz = gxyz.reshape(_B, _NPOINT, _NSAMPLE, 3)
    gfeat = gfeat.reshape(_B, _NPOINT, _NSAMPLE, _C)
    grouped = jnp.concatenate([gxyz, gfeat], axis=-1)
    return (newxyz, inds, grouped, isPainted)


# speedup vs baseline: 1.0102x; 1.0102x over previous
import numpy as np
import jax
import jax.numpy as jnp
from jax import lax
from jax.experimental import pallas as pl
from jax.experimental.pallas import tpu as pltpu
from jax.experimental.pallas import tpu_sc as plsc

_B = 4
_N = 16384
_C = 128
_NPOINT = 1024
_NSAMPLE = 64
_RADIUS = np.float32(0.4)
_R2 = np.float32(0.4 * 0.4)

_SIDE = 128

_NW = 32
_SPB = _NW // _B
_SEEDS_PER_W = _NPOINT // _SPB
_NCHUNK = _N // 16
_UNROLL = 16


def _fps_body(xyzt_ref, inds_ref, dists_ref):
    X = xyzt_ref[:, 0, :, :]
    Y = xyzt_ref[:, 1, :, :]
    Z = xyzt_ref[:, 2, :, :]
    row = lax.broadcasted_iota(jnp.int32, (_SIDE, _SIDE), 0)
    col = lax.broadcasted_iota(jnp.int32, (_SIDE, _SIDE), 1)
    iota2 = (row * _SIDE + col)[None]
    cols = lax.broadcasted_iota(jnp.int32, (1, _NPOINT), 1)
    l128 = lax.broadcasted_iota(jnp.int32, (1, _SIDE), 1)
    big = jnp.int32(1 << 30)
    dists_ref[...] = jnp.full((_B, _SIDE, _SIDE), 1e10, dtype=jnp.float32)

    def body(i, state):
        fara, farb, inds = state
        farcat = jnp.concatenate([fara, farb], axis=0).reshape(_B, 1)
        inds = inds + jnp.where(cols == i, farcat, 0)
        new_fars = []
        for g, far in ((0, fara), (1, farb)):
            sl = slice(2 * g, 2 * g + 2)
            Xg = X[sl]
            Yg = Y[sl]
            Zg = Z[sl]
            oh = iota2 == far
            cxyz = jnp.sum(jnp.where(oh[:, None], xyzt_ref[sl], 0.0),
                           axis=(2, 3), keepdims=True)
            cx = cxyz[:, 0]
            cy = cxyz[:, 1]
            cz = cxyz[:, 2]
            dx = Xg - cx
            dy = Yg - cy
            dz = Zg - cz
            d = dx * dx + dy * dy
            d = d + dz * dz
            dists = jnp.minimum(dists_ref[sl], d)
            dists_ref[sl] = dists
            m = jnp.max(dists, axis=(1, 2), keepdims=True)
            fnew = jnp.min(jnp.where(dists == m, iota2, big),
                           axis=(1, 2), keepdims=True)
            new_fars.append(fnew.astype(jnp.int32))
        return (new_fars[0], new_fars[1], inds)

    far0 = jnp.zeros((2, 1, 1), dtype=jnp.int32)
    inds0 = jnp.zeros((_B, _NPOINT), dtype=jnp.int32)
    _, _, inds_final = lax.fori_loop(0, _NPOINT, body, (far0, far0, inds0))
    inds_ref[...] = inds_final


def _sc_group_body(xyzt_hbm, inds_hbm, feat_hbm,
                   newxyz_hbm, gxyz_hbm, gfeat_hbm,
                   xv, yv, zv, indsv, sxv, syv, szv, nxv,
                   grpbuf, gidxa, gidxb, xyza, xyzb, rowa, rowb, sem, osem):
    cid = lax.axis_index("c")
    sid = lax.axis_index("s")
    wid = sid * 2 + cid
    b = wid // _SPB
    s0 = (wid % _SPB) * _SEEDS_PER_W
    bN = b * _N

    pltpu.sync_copy(xyzt_hbm.at[pl.ds((b * 3 + 0) * _N, _N)], xv)
    pltpu.sync_copy(xyzt_hbm.at[pl.ds((b * 3 + 1) * _N, _N)], yv)
    pltpu.sync_copy(xyzt_hbm.at[pl.ds((b * 3 + 2) * _N, _N)], zv)
    pltpu.sync_copy(inds_hbm.at[pl.ds(b * _NPOINT + s0, _SEEDS_PER_W)], indsv)

    iota16 = lax.broadcasted_iota(jnp.int32, (16,), 0)
    c0 = jnp.zeros((16,), jnp.int32)
    c1 = jnp.full((16,), 1, jnp.int32)
    c2 = jnp.full((16,), 2, jnp.int32)

    sxv[pl.ds(_SEEDS_PER_W, 16)] = jnp.zeros((16,), jnp.float32)
    syv[pl.ds(_SEEDS_PER_W, 16)] = jnp.zeros((16,), jnp.float32)
    szv[pl.ds(_SEEDS_PER_W, 16)] = jnp.zeros((16,), jnp.float32)

    for q in range(_SEEDS_PER_W // 16):
        iv = indsv[pl.ds(q * 16, 16)]
        sx = plsc.load_gather(xv, [iv])
        sy = plsc.load_gather(yv, [iv])
        sz = plsc.load_gather(zv, [iv])
        sxv[pl.ds(q * 16, 16)] = sx
        syv[pl.ds(q * 16, 16)] = sy
        szv[pl.ds(q * 16, 16)] = sz
        kvec3 = (iota16 + q * 16) * 3
        plsc.store_scatter(nxv, [kvec3 + c0], sx)
        plsc.store_scatter(nxv, [kvec3 + c1], sy)
        plsc.store_scatter(nxv, [kvec3 + c2], sz)
    pltpu.sync_copy(nxv, newxyz_hbm.at[pl.ds((b * _NPOINT + s0) * 3, _SEEDS_PER_W * 3)])

    def build(t, gidx_ref, xyzb_ref):
        qb = (t // 16) * 16
        lane = t - qb
        lm = iota16 == lane
        sxt = jnp.sum(jnp.where(lm, sxv[pl.ds(qb, 16)], 0.0))
        syt = jnp.sum(jnp.where(lm, syv[pl.ds(qb, 16)], 0.0))
        szt = jnp.sum(jnp.where(lm, szv[pl.ds(qb, 16)], 0.0))

        def cond(st):
            j, offv = st
            return jnp.logical_and(j < _NCHUNK,
                                   jnp.all(offv < _NSAMPLE))

        def sbody(st):
            j, offv = st
            acc = offv - 1
            for u in range(_UNROLL):
                base = (j + u) * 16
                xs = xv[pl.ds(base, 16)]
                ys = yv[pl.ds(base, 16)]
                zs = zv[pl.ds(base, 16)]
                dx = xs - sxt
                dy = ys - syt
                dz = zs - szt
                d = dx * dx + dy * dy
                d = d + dz * dz
                m = d <= _R2
                pos = plsc.cumsum(m.astype(jnp.int32)) + acc
                plsc.store_scatter(grpbuf, [pos], iota16 + base, mask=m)
                acc = acc + plsc.all_reduce_population_count(m)
            return (j + _UNROLL, acc + 1)

        _, offv = lax.while_loop(
            cond, sbody, (jnp.int32(0), jnp.zeros((16,), jnp.int32)))
        total = jnp.max(offv)

        f16 = grpbuf[pl.ds(0, 16)]
        first = jnp.sum(jnp.where(iota16 == 0, f16, 0))

        for c in range(_NSAMPLE // 16):
            kvec = iota16 + c * 16
            g = grpbuf[pl.ds(c * 16, 16)]
            g = jnp.where(kvec < total, g, first)
            gx = plsc.load_gather(xv, [g])
            gy = plsc.load_gather(yv, [g])
            gz = plsc.load_gather(zv, [g])
            kvec3 = kvec * 3
            plsc.store_scatter(xyzb_ref, [kvec3 + c0], (gx - sxt) / _RADIUS)
            plsc.store_scatter(xyzb_ref, [kvec3 + c1], (gy - syt) / _RADIUS)
            plsc.store_scatter(xyzb_ref, [kvec3 + c2], (gz - szt) / _RADIUS)
            gidx_ref[pl.ds(c * 16, 16)] = g + bN

    def feat_dst(t):
        s = b * _NPOINT + s0 + t
        return gfeat_hbm.at[pl.ds(s * _NSAMPLE, _NSAMPLE), :]

    def xyz_out(t, xyzb_ref):
        s = b * _NPOINT + s0 + t
        pltpu.sync_copy(xyzb_ref, gxyz_hbm.at[pl.ds(s * _NSAMPLE * 3, _NSAMPLE * 3)])

    build(0, gidxa, xyza)
    pltpu.async_copy(feat_hbm.at[gidxa], rowa, sem)
    pltpu.async_copy(feat_hbm.at[pl.ds(0, _NSAMPLE), :], rowb, osem)

    def pair(i, carry):
        t0 = 2 * i
        build(t0 + 1, gidxb, xyzb)
        pltpu.make_async_copy(feat_hbm.at[gidxa], rowa, sem).wait()
        pltpu.make_async_copy(rowb, feat_dst(t0 + 1), osem).wait()
        pltpu.async_copy(feat_hbm.at[gidxb], rowb, sem)
        pltpu.async_copy(rowa, feat_dst(t0), osem)
        xyz_out(t0, xyza)
        build(t0 + 2, gidxa, xyza)
        pltpu.make_async_copy(feat_hbm.at[gidxb], rowb, sem).wait()
        pltpu.make_async_copy(rowa, feat_dst(t0), osem).wait()
        pltpu.async_copy(feat_hbm.at[gidxa], rowa, sem)
        pltpu.async_copy(rowb, feat_dst(t0 + 1), osem)
        xyz_out(t0 + 1, xyzb)
        return carry

    lax.fori_loop(0, _SEEDS_PER_W // 2, pair, jnp.int32(0))
    pltpu.make_async_copy(rowb, feat_dst(_SEEDS_PER_W - 1), osem).wait()
    pltpu.make_async_copy(feat_hbm.at[gidxa], rowa, sem).wait()


def kernel(xyz, isPainted, features):
    xyzt = xyz.transpose(0, 2, 1).reshape(_B, 3, _SIDE, _SIDE)

    inds = pl.pallas_call(
        _fps_body,
        out_shape=jax.ShapeDtypeStruct((_B, _NPOINT), jnp.int32),
        scratch_shapes=[pltpu.VMEM((_B, _SIDE, _SIDE), jnp.float32)],
    )(xyzt)

    xyztflat = xyzt.reshape(_B * 3 * _N)
    indsflat = inds.reshape(_B * _NPOINT)
    featflat = features.reshape(_B * _N, _C)

    mesh = plsc.VectorSubcoreMesh(core_axis_name="c", subcore_axis_name="s")
    newxyz, gxyz, gfeat = pl.kernel(
        _sc_group_body,
        out_type=[
            jax.ShapeDtypeStruct((_B * _NPOINT * 3,), jnp.float32),
            jax.ShapeDtypeStruct((_B * _NPOINT * _NSAMPLE * 3,), jnp.float32),
            jax.ShapeDtypeStruct((_B * _NPOINT * _NSAMPLE, _C), jnp.float32),
        ],
        mesh=mesh,
        compiler_params=pltpu.CompilerParams(needs_layout_passes=False),
        scratch_types=[
            pltpu.VMEM((_N,), jnp.float32),
            pltpu.VMEM((_N,), jnp.float32),
            pltpu.VMEM((_N,), jnp.float32),
            pltpu.VMEM((_SEEDS_PER_W,), jnp.int32),
            pltpu.VMEM((_SEEDS_PER_W + 16,), jnp.float32),
            pltpu.VMEM((_SEEDS_PER_W + 16,), jnp.float32),
            pltpu.VMEM((_SEEDS_PER_W + 16,), jnp.float32),
            pltpu.VMEM((_SEEDS_PER_W * 3,), jnp.float32),
            pltpu.VMEM((320,), jnp.int32),
            pltpu.VMEM((_NSAMPLE,), jnp.int32),
            pltpu.VMEM((_NSAMPLE,), jnp.int32),
            pltpu.VMEM((_NSAMPLE * 3,), jnp.float32),
            pltpu.VMEM((_NSAMPLE * 3,), jnp.float32),
            pltpu.VMEM((_NSAMPLE, _C), jnp.float32),
            pltpu.VMEM((_NSAMPLE, _C), jnp.float32),
            pltpu.SemaphoreType.DMA,
            pltpu.SemaphoreType.DMA,
        ],
    )(xyztflat, indsflat, featflat)

    newxyz = newxyz.reshape(_B, _NPOINT, 3)
    gxyz = gxyz.reshape(_B, _NPOINT, _NSAMPLE, 3)
    gfeat = gfeat.reshape(_B, _NPOINT, _NSAMPLE, _C)
    grouped = jnp.concatenate([gxyz, gfeat], axis=-1)
    return (newxyz, inds, grouped, isPainted)
